# Initial kernel scaffold; baseline (speedup 1.0000x reference)
import functools

import jax
import jax.numpy as jnp
from jax import lax
from jax.experimental import pallas as pl
from jax.experimental.pallas import tpu as pltpu
from jax.experimental.pallas import tpu_sc as plsc

B = 4
S = 2048
D = 768
OFFSET = 2
N = B * S

NC, NS = 2, 16
NW = NC * NS
RPW = N // NW
CH = 32
NCH = RPW // CH
PPW = S // NW

_mesh = plsc.VectorSubcoreMesh(core_axis_name="c", subcore_axis_name="s")


@functools.partial(
    pl.kernel,
    mesh=_mesh,
    out_type=[
        jax.ShapeDtypeStruct((N, D), jnp.float32),
        jax.ShapeDtypeStruct((B, S, D), jnp.float32),
    ],
    scratch_types=[
        pltpu.VMEM((NCH, CH), jnp.int32),
        pltpu.VMEM((CH, D), jnp.float32),
        pltpu.VMEM((CH, D), jnp.float32),
        pltpu.VMEM((PPW, D), jnp.float32),
        pltpu.SemaphoreType.DMA,
        pltpu.SemaphoreType.DMA,
    ],
)
def _emb_kernel(ids_hbm, table_hbm, ptab_hbm, out_tok, out_pos,
                idx_v, buf0, buf1, pos_buf, sem0, sem1):
    wid = lax.axis_index("s") * NC + lax.axis_index("c")
    base = wid * RPW

    pltpu.sync_copy(ids_hbm.at[pl.ds(wid * NCH, NCH)], idx_v)

    bufs = (buf0, buf1)
    sems = (sem0, sem1)

    pltpu.make_async_copy(table_hbm.at[idx_v.at[0]], bufs[0], sems[0]).start()
    for c in range(NCH):
        if c + 1 < NCH:
            pltpu.make_async_copy(
                table_hbm.at[idx_v.at[c + 1]],
                bufs[(c + 1) % 2],
                sems[(c + 1) % 2],
            ).start()
        pltpu.make_async_copy(
            table_hbm.at[idx_v.at[c]], bufs[c % 2], sems[c % 2]
        ).wait()
        pltpu.sync_copy(bufs[c % 2], out_tok.at[pl.ds(base + c * CH, CH)])

    pbase = wid * PPW
    pltpu.sync_copy(ptab_hbm.at[pl.ds(OFFSET + pbase, PPW)], pos_buf)
    for b in range(B):
        pltpu.sync_copy(pos_buf, out_pos.at[b].at[pl.ds(pbase, PPW)])


def kernel(input_ids, embed_tokens, embed_positions):
    ids = input_ids.reshape(NW * NCH, CH).astype(jnp.int32)
    tok_flat, pos_embeds = _emb_kernel(ids, embed_tokens, embed_positions)
    inputs_embeds = tok_flat.reshape(B, S, D)
    attention_mask = jnp.ones((B, S), dtype=jnp.float32)
    return (inputs_embeds, attention_mask, pos_embeds)


# speedup vs baseline: 1.1808x; 1.1808x over previous
import functools

import jax
import jax.numpy as jnp
from jax import lax
from jax.experimental import pallas as pl
from jax.experimental.pallas import tpu as pltpu
from jax.experimental.pallas import tpu_sc as plsc

B = 4
S = 2048
D = 768
OFFSET = 2
N = B * S

NC, NS = 2, 16
NW = NC * NS
RPW = N // NW
CH = 32
NCH = RPW // CH
PPW = S // NW

_mesh = plsc.VectorSubcoreMesh(core_axis_name="c", subcore_axis_name="s")


@functools.partial(
    pl.kernel,
    mesh=_mesh,
    out_type=[
        jax.ShapeDtypeStruct((N, D), jnp.float32),
        jax.ShapeDtypeStruct((B * S * D,), jnp.float32),
    ],
    scratch_types=[
        pltpu.VMEM((NCH, CH), jnp.int32),
        pltpu.VMEM((CH, D), jnp.float32),
        pltpu.VMEM((CH, D), jnp.float32),
        pltpu.VMEM((PPW * D,), jnp.float32),
        pltpu.SemaphoreType.DMA,
        pltpu.SemaphoreType.DMA,
    ],
)
def _emb_kernel(ids_hbm, table_hbm, ptab_hbm, out_tok, out_pos,
                idx_v, buf0, buf1, pos_buf, sem0, sem1):
    wid = lax.axis_index("s") * NC + lax.axis_index("c")
    base = wid * RPW

    pltpu.sync_copy(ids_hbm.at[pl.ds(wid * NCH, NCH)], idx_v)

    bufs = (buf0, buf1)
    sems = (sem0, sem1)

    pltpu.make_async_copy(table_hbm.at[idx_v.at[0]], bufs[0], sems[0]).start()
    for c in range(NCH):
        if c + 1 < NCH:
            pltpu.make_async_copy(
                table_hbm.at[idx_v.at[c + 1]],
                bufs[(c + 1) % 2],
                sems[(c + 1) % 2],
            ).start()
        pltpu.make_async_copy(
            table_hbm.at[idx_v.at[c]], bufs[c % 2], sems[c % 2]
        ).wait()
        pltpu.sync_copy(bufs[c % 2], out_tok.at[pl.ds(base + c * CH, CH)])

    pbase = wid * PPW
    pltpu.sync_copy(ptab_hbm.at[pl.ds((OFFSET + pbase) * D, PPW * D)], pos_buf)
    for b in range(B):
        pltpu.sync_copy(pos_buf, out_pos.at[pl.ds((b * S + pbase) * D, PPW * D)])


def kernel(input_ids, embed_tokens, embed_positions):
    ids = input_ids.reshape(NW * NCH, CH).astype(jnp.int32)
    tok_flat, pos_flat = _emb_kernel(ids, embed_tokens,
                                     embed_positions.reshape(-1))
    inputs_embeds = tok_flat.reshape(B, S, D)
    pos_embeds = pos_flat.reshape(B, S, D)
    attention_mask = jnp.ones((B, S), dtype=jnp.float32)
    return (inputs_embeds, attention_mask, pos_embeds)
